# baseline (speedup 1.0000x reference)
import functools

import jax
import jax.numpy as jnp
from jax import lax
from jax.experimental import pallas as pl
from jax.experimental.pallas import tpu as pltpu
from jax.experimental.pallas import tpu_sc as plsc

NUM_RELATIONS = 1000
REL_PAD = 1024
DIM = 32
BATCH = 16384
NC = 2
NS = 16
NW = NC * NS
B_PER_W = BATCH // NW
E_WORDS = B_PER_W * DIM
W_WORDS = REL_PAD * DIM
TI_STRIDE = BATCH * 8


def _native_flat(x, rows):
    a = jnp.reshape(jnp.transpose(x), (4, 8, rows // 128, 128))
    return jnp.reshape(jnp.transpose(a, (0, 2, 1, 3)), (rows * DIM,))


@functools.partial(
    pl.kernel,
    out_type=jax.ShapeDtypeStruct((BATCH,), jnp.float32),
    mesh=plsc.VectorSubcoreMesh(core_axis_name="c", subcore_axis_name="s"),
    compiler_params=pltpu.CompilerParams(
        needs_layout_passes=False, use_tc_tiling_on_sc=False,
        skip_device_barrier=True, disable_bounds_checks=True,
        disable_semaphore_checks=True),
    scratch_types=[
        pltpu.VMEM((B_PER_W,), jnp.int32),
        pltpu.VMEM((E_WORDS,), jnp.float32),
        pltpu.VMEM((W_WORDS,), jnp.float32),
        pltpu.VMEM((E_WORDS,), jnp.float32),
        pltpu.VMEM((B_PER_W,), jnp.float32),
        pltpu.SemaphoreType.DMA,
        pltpu.SemaphoreType.DMA,
        pltpu.SemaphoreType.DMA,
        pltpu.SemaphoreType.DMA,
    ],
)
def _dist_mult(h_hbm, r_hbm, t_hbm, w_hbm, out_hbm,
               idx_v, h_v, w_v, t_v, out_v, *sems):
    wid = lax.axis_index("s") * NC + lax.axis_index("c")
    base = wid * B_PER_W

    copies = []
    for ti in range(4):
        src = ti * TI_STRIDE + wid * 4096
        copies.append([
            pltpu.async_copy(w_hbm.at[pl.ds(ti * 8192, 8192)],
                             w_v.at[pl.ds(ti * 8192, 8192)], sems[ti]),
            pltpu.async_copy(h_hbm.at[pl.ds(src, 4096)],
                             h_v.at[pl.ds(ti * 4096, 4096)], sems[ti]),
            pltpu.async_copy(t_hbm.at[pl.ds(src, 4096)],
                             t_v.at[pl.ds(ti * 4096, 4096)], sems[ti]),
        ])
    pltpu.sync_copy(r_hbm.at[pl.ds(base, B_PER_W)], idx_v)

    def make_pass(ti):
        def group(g):
            r16 = idx_v[pl.ds(g * 16, 16)]
            wrow = ((r16 >> 7) << 10) + (r16 & 127)
            ebase = (g // 8) * 1024 + (g % 8) * 16
            accs = [jnp.zeros((16,), jnp.float32) for _ in range(2)]
            for dd in range(8):
                eoff = ti * 4096 + dd * 128 + ebase
                woff = ti * 8192 + dd * 128
                h = h_v[pl.ds(eoff, 16)]
                t = t_v[pl.ds(eoff, 16)]
                w = plsc.load_gather(w_v, [wrow + woff])
                accs[dd % 2] = accs[dd % 2] + h * w * t
            acc = accs[0] + accs[1]
            ds = pl.ds(g * 16, 16)
            if ti == 0:
                out_v[ds] = acc
            else:
                out_v[ds] = out_v[ds] + acc
        return group

    for ti in range(4):
        for cp in copies[ti]:
            cp.wait()
        plsc.parallel_loop(0, B_PER_W // 16, unroll=4)(make_pass(ti))
    pltpu.sync_copy(out_v, out_hbm.at[pl.ds(base, B_PER_W)])


def kernel(e_h, r, e_t, rel_weight):
    w_pad = jnp.zeros((REL_PAD, DIM), jnp.float32).at[:NUM_RELATIONS].set(
        rel_weight)
    return _dist_mult(
        _native_flat(e_h, BATCH),
        r.astype(jnp.int32),
        _native_flat(e_t, BATCH),
        _native_flat(w_pad, REL_PAD),
    )


# speedup vs baseline: 1.0315x; 1.0315x over previous
import functools

import jax
import jax.numpy as jnp
from jax import lax
from jax.experimental import pallas as pl
from jax.experimental.pallas import tpu as pltpu
from jax.experimental.pallas import tpu_sc as plsc

NUM_RELATIONS = 1000
REL_PAD = 1024
DIM = 32
BATCH = 16384
NC = 2
NS = 16
NW = NC * NS
B_PER_W = BATCH // NW
E_WORDS = B_PER_W * DIM
W_WORDS = REL_PAD * DIM
TI_STRIDE = BATCH * 8


def _native_flat(x, rows):
    a = jnp.reshape(jnp.transpose(x), (4, 8, rows // 128, 128))
    return jnp.reshape(jnp.transpose(a, (0, 2, 1, 3)), (rows * DIM,))


@functools.partial(
    pl.kernel,
    out_type=jax.ShapeDtypeStruct((BATCH,), jnp.float32),
    mesh=plsc.VectorSubcoreMesh(core_axis_name="c", subcore_axis_name="s"),
    compiler_params=pltpu.CompilerParams(
        needs_layout_passes=False, use_tc_tiling_on_sc=False,
        skip_device_barrier=True, disable_bounds_checks=True,
        disable_semaphore_checks=True),
    scratch_types=[
        pltpu.VMEM((B_PER_W,), jnp.int32),
        pltpu.VMEM((E_WORDS,), jnp.float32),
        pltpu.VMEM((W_WORDS,), jnp.float32),
        pltpu.VMEM((E_WORDS,), jnp.float32),
        pltpu.VMEM((B_PER_W,), jnp.float32),
        pltpu.SemaphoreType.DMA,
        pltpu.SemaphoreType.DMA,
        pltpu.SemaphoreType.DMA,
        pltpu.SemaphoreType.DMA,
    ],
)
def _dist_mult(h_hbm, r_hbm, t_hbm, w_hbm, out_hbm,
               idx_v, h_v, w_v, t_v, out_v, *sems):
    wid = lax.axis_index("s") * NC + lax.axis_index("c")
    base = wid * B_PER_W

    copies = []
    for ti in range(4):
        src = ti * TI_STRIDE + wid * 4096
        copies.append([
            pltpu.async_copy(w_hbm.at[pl.ds(ti * 8192, 8192)],
                             w_v.at[pl.ds(ti * 8192, 8192)], sems[ti]),
            pltpu.async_copy(h_hbm.at[pl.ds(src, 4096)],
                             h_v.at[pl.ds(ti * 4096, 4096)], sems[ti]),
            pltpu.async_copy(t_hbm.at[pl.ds(src, 4096)],
                             t_v.at[pl.ds(ti * 4096, 4096)], sems[ti]),
        ])
    pltpu.sync_copy(r_hbm.at[pl.ds(base, B_PER_W)], idx_v)

    def make_pass(ti):
        def group(g):
            r16 = idx_v[pl.ds(g * 16, 16)]
            wrow = ((r16 >> 7) << 10) + (r16 & 127)
            ebase = (g // 8) * 1024 + (g % 8) * 16
            accs = [jnp.zeros((16,), jnp.float32) for _ in range(2)]
            for dd in range(8):
                eoff = ti * 4096 + dd * 128 + ebase
                woff = ti * 8192 + dd * 128
                h = h_v[pl.ds(eoff, 16)]
                t = t_v[pl.ds(eoff, 16)]
                w = plsc.load_gather(w_v, [wrow + woff])
                accs[dd % 2] = accs[dd % 2] + h * w * t
            acc = accs[0] + accs[1]
            ds = pl.ds(g * 16, 16)
            if ti == 0:
                out_v[ds] = acc
            else:
                out_v[ds] = out_v[ds] + acc
        return group

    for ti in range(4):
        for cp in copies[ti]:
            cp.wait()
        plsc.parallel_loop(0, B_PER_W // 16, unroll=2)(make_pass(ti))
    pltpu.sync_copy(out_v, out_hbm.at[pl.ds(base, B_PER_W)])


def kernel(e_h, r, e_t, rel_weight):
    w_pad = jnp.zeros((REL_PAD, DIM), jnp.float32).at[:NUM_RELATIONS].set(
        rel_weight)
    return _dist_mult(
        _native_flat(e_h, BATCH),
        r.astype(jnp.int32),
        _native_flat(e_t, BATCH),
        _native_flat(w_pad, REL_PAD),
    )
